# baseline (speedup 1.0000x reference)
import functools

import jax
import jax.numpy as jnp
from jax import lax
from jax.experimental import pallas as pl
from jax.experimental.pallas import tpu as pltpu
from jax.experimental.pallas import tpu_sc as plsc

B = 16384
D = 32
WIDE = 128
NC = 2
NS = 16
NW = NC * NS
BPW = B // NW
CH = 128


def _gather_body(u_tab, i_tab, uw_idx, iw_idx, ue_out, ie_out,
                 idx_v, rows_v, sem):
    wid = lax.axis_index("s") * NC + lax.axis_index("c")
    base = wid * BPW
    for tab, idx_hbm, out in ((u_tab, uw_idx, ue_out), (i_tab, iw_idx, ie_out)):
        pltpu.sync_copy(idx_hbm.at[pl.ds(base, BPW)], idx_v)
        for j in range(BPW // CH):
            sl = pl.ds(j * CH, CH)
            pltpu.async_copy(tab.at[idx_v.at[sl]], rows_v.at[sl], sem)
        pltpu.make_async_copy(tab.at[idx_v], rows_v, sem).wait()
        pltpu.sync_copy(rows_v, out.at[pl.ds(base, BPW)])


_gather = functools.partial(
    pl.kernel,
    mesh=plsc.VectorSubcoreMesh(core_axis_name="c", subcore_axis_name="s"),
    out_type=(
        jax.ShapeDtypeStruct((B, WIDE), jnp.float32),
        jax.ShapeDtypeStruct((B, WIDE), jnp.float32),
    ),
    scratch_types=[
        pltpu.VMEM((BPW,), jnp.int32),
        pltpu.VMEM((BPW, WIDE), jnp.float32),
        pltpu.SemaphoreType.DMA,
    ],
)(_gather_body)


BLK = 2048


def _select(w, s):
    x = jnp.where(s == 0, 1.0, 0.0) * w[:, 0 * D:1 * D]
    x += jnp.where(s == 1, 1.0, 0.0) * w[:, 1 * D:2 * D]
    x += jnp.where(s == 2, 1.0, 0.0) * w[:, 2 * D:3 * D]
    x += jnp.where(s == 3, 1.0, 0.0) * w[:, 3 * D:4 * D]
    return x


def _mlp_body(uw, iw, us, isel, w1u, w1i, b1, w2, b2, w3, b3, out):
    ue = _select(uw[...], us[...])
    ie = _select(iw[...], isel[...])
    h = jnp.dot(ue, w1u[...], preferred_element_type=jnp.float32)
    h = h + jnp.dot(ie, w1i[...], preferred_element_type=jnp.float32)
    h = jnp.maximum(h + b1[...], 0.0)
    h = jnp.dot(h, w2[...], preferred_element_type=jnp.float32)
    h = jnp.maximum(h + b2[...], 0.0)
    s = jnp.dot(h, w3[...], preferred_element_type=jnp.float32) + b3[...]
    out[...] = jax.nn.sigmoid(s)


_mlp = pl.pallas_call(
    _mlp_body,
    grid=(B // BLK,),
    in_specs=[
        pl.BlockSpec((BLK, WIDE), lambda b: (b, 0)),
        pl.BlockSpec((BLK, WIDE), lambda b: (b, 0)),
        pl.BlockSpec((BLK, 1), lambda b: (b, 0)),
        pl.BlockSpec((BLK, 1), lambda b: (b, 0)),
        pl.BlockSpec((D, 64), lambda b: (0, 0)),
        pl.BlockSpec((D, 64), lambda b: (0, 0)),
        pl.BlockSpec((1, 64), lambda b: (0, 0)),
        pl.BlockSpec((64, 32), lambda b: (0, 0)),
        pl.BlockSpec((1, 32), lambda b: (0, 0)),
        pl.BlockSpec((32, 1), lambda b: (0, 0)),
        pl.BlockSpec((1, 1), lambda b: (0, 0)),
    ],
    out_specs=pl.BlockSpec((BLK, 1), lambda b: (b, 0)),
    out_shape=jax.ShapeDtypeStruct((B, 1), jnp.float32),
)


def kernel(u, i, user_emb, item_emb, W1, b1, W2, b2, W3, b3):
    u32 = u.astype(jnp.int32)
    i32 = i.astype(jnp.int32)
    uw_idx = u32 >> 2
    iw_idx = i32 >> 2
    u_sel = (u32 & 3).reshape(B, 1)
    i_sel = (i32 & 3).reshape(B, 1)
    u_tab = user_emb.reshape(250000, WIDE)
    i_tab = item_emb.reshape(250000, WIDE)
    uw, iw = _gather(u_tab, i_tab, uw_idx, iw_idx)
    w1u = W1[:, :D].T
    w1i = W1[:, D:].T
    out = _mlp(uw, iw, u_sel, i_sel, w1u, w1i, b1.reshape(1, -1), W2.T,
               b2.reshape(1, -1), W3.T, b3.reshape(1, 1))
    return out.reshape(B)


# speedup vs baseline: 1.5234x; 1.5234x over previous
import functools

import jax
import jax.numpy as jnp
from jax import lax
from jax.experimental import pallas as pl
from jax.experimental.pallas import tpu as pltpu
from jax.experimental.pallas import tpu_sc as plsc

B = 16384
D = 32
NC = 2
NS = 16
NW = NC * NS
BPW = B // NW


def _gather_body(u_tab, i_tab, u_idx, i_idx, ue_out, ie_out,
                 idx_s, idx_v, rows_v, sem):
    wid = lax.axis_index("s") * NC + lax.axis_index("c")
    base = wid * BPW
    for tab, idx_hbm, out in ((u_tab, u_idx, ue_out), (i_tab, i_idx, ie_out)):
        pltpu.sync_copy(idx_hbm.at[pl.ds(base, BPW)], idx_v)

        def body(c, carry, tab=tab):
            vec = idx_v[pl.ds(c * 16, 16)]
            for k in range(16):
                pltpu.async_copy(tab.at[vec[k]], rows_v.at[c * 16 + k], sem)
            return carry

        lax.fori_loop(0, BPW // 16, body, 0)
        pltpu.make_async_copy(tab.at[pl.ds(0, BPW)], rows_v, sem).wait()
        pltpu.sync_copy(rows_v, out.at[pl.ds(base, BPW)])


_gather = functools.partial(
    pl.kernel,
    mesh=plsc.VectorSubcoreMesh(core_axis_name="c", subcore_axis_name="s"),
    out_type=(
        jax.ShapeDtypeStruct((B, D), jnp.float32),
        jax.ShapeDtypeStruct((B, D), jnp.float32),
    ),
    scratch_types=[
        pltpu.SMEM((BPW,), jnp.int32),
        pltpu.VMEM((BPW,), jnp.int32),
        pltpu.VMEM((BPW, D), jnp.float32),
        pltpu.SemaphoreType.DMA,
    ],
)(_gather_body)


BLK = 2048


def _mlp_body(ue, ie, w1u, w1i, b1, w2, b2, w3, b3, out):
    h = jnp.dot(ue[...], w1u[...], preferred_element_type=jnp.float32)
    h = h + jnp.dot(ie[...], w1i[...], preferred_element_type=jnp.float32)
    h = jnp.maximum(h + b1[...], 0.0)
    h = jnp.dot(h, w2[...], preferred_element_type=jnp.float32)
    h = jnp.maximum(h + b2[...], 0.0)
    s = jnp.dot(h, w3[...], preferred_element_type=jnp.float32) + b3[...]
    out[...] = jax.nn.sigmoid(s)


_mlp = pl.pallas_call(
    _mlp_body,
    grid=(B // BLK,),
    in_specs=[
        pl.BlockSpec((BLK, D), lambda b: (b, 0)),
        pl.BlockSpec((BLK, D), lambda b: (b, 0)),
        pl.BlockSpec((D, 64), lambda b: (0, 0)),
        pl.BlockSpec((D, 64), lambda b: (0, 0)),
        pl.BlockSpec((1, 64), lambda b: (0, 0)),
        pl.BlockSpec((64, 32), lambda b: (0, 0)),
        pl.BlockSpec((1, 32), lambda b: (0, 0)),
        pl.BlockSpec((32, 1), lambda b: (0, 0)),
        pl.BlockSpec((1, 1), lambda b: (0, 0)),
    ],
    out_specs=pl.BlockSpec((BLK, 1), lambda b: (b, 0)),
    out_shape=jax.ShapeDtypeStruct((B, 1), jnp.float32),
)


def kernel(u, i, user_emb, item_emb, W1, b1, W2, b2, W3, b3):
    u32 = u.astype(jnp.int32)
    i32 = i.astype(jnp.int32)
    ue, ie = _gather(user_emb, item_emb, u32, i32)
    w1u = W1[:, :D].T
    w1i = W1[:, D:].T
    out = _mlp(ue, ie, w1u, w1i, b1.reshape(1, -1), W2.T, b2.reshape(1, -1),
               W3.T, b3.reshape(1, 1))
    return out.reshape(B)
